# baseline (speedup 1.0000x reference)
import functools

import jax
import jax.numpy as jnp
from jax import lax
from jax.experimental import pallas as pl
from jax.experimental.pallas import tpu as pltpu
from jax.experimental.pallas import tpu_sc as plsc

B = 16384
L = 50
EMBED = 64
NC, NS = 2, 16
NW = NC * NS
B_PER_W = B // NW
NBUF = 8
NGROUP = B_PER_W // NBUF

_mesh = plsc.VectorSubcoreMesh(core_axis_name="c", subcore_axis_name="s")


@functools.partial(
    pl.kernel,
    mesh=_mesh,
    out_type=jax.ShapeDtypeStruct((B, L, EMBED), jnp.float32),
    scratch_types=[
        pltpu.VMEM((B_PER_W, L), jnp.int32),
        pltpu.VMEM((NBUF, L, EMBED), jnp.float32),
        pltpu.SemaphoreType.DMA((NBUF,)),
        pltpu.SemaphoreType.DMA((NBUF,)),
    ],
    compiler_params=pltpu.CompilerParams(use_tc_tiling_on_sc=False),
)
def _gather_all(idx_hbm, table_hbm, out_hbm, idx_v, rows_v, semg, sems):
    wid = lax.axis_index("s") * NC + lax.axis_index("c")
    base = wid * B_PER_W

    pltpu.sync_copy(idx_hbm.at[pl.ds(base, B_PER_W)], idx_v)

    def group(g, carry):
        for b in range(NBUF):
            j = g * NBUF + b

            @pl.when(g > 0)
            def _wait_store():
                pltpu.make_async_copy(
                    rows_v.at[b], out_hbm.at[0], sems.at[b]
                ).wait()

            pltpu.async_copy(
                table_hbm.at[idx_v.at[j]], rows_v.at[b], semg.at[b]
            )
        for b in range(NBUF):
            j = g * NBUF + b
            pltpu.make_async_copy(
                table_hbm.at[idx_v.at[j]], rows_v.at[b], semg.at[b]
            ).wait()
            pltpu.async_copy(
                rows_v.at[b], out_hbm.at[base + j], sems.at[b]
            )
        return carry

    lax.fori_loop(0, NGROUP, group, 0)

    for b in range(NBUF):
        pltpu.make_async_copy(
            rows_v.at[b], out_hbm.at[0], sems.at[b]
        ).wait()


def kernel(input, table):
    return _gather_all(input, table)


# speedup vs baseline: 1.0065x; 1.0065x over previous
import functools

import jax
import jax.numpy as jnp
from jax import lax
from jax.experimental import pallas as pl
from jax.experimental.pallas import tpu as pltpu
from jax.experimental.pallas import tpu_sc as plsc

B = 16384
L = 50
EMBED = 64
NC, NS = 2, 16
NW = NC * NS
B_PER_W = B // NW
CHUNK = 128
KPL = B_PER_W // CHUNK
NBUF = 8
NGROUP = L * KPL // NBUF

_mesh = plsc.VectorSubcoreMesh(core_axis_name="c", subcore_axis_name="s")


@functools.partial(
    pl.kernel,
    mesh=_mesh,
    out_type=jax.ShapeDtypeStruct((B, L, EMBED), jnp.float32),
    scratch_types=[
        pltpu.VMEM((L, B_PER_W), jnp.int32),
        pltpu.VMEM((NBUF, CHUNK, EMBED), jnp.float32),
        pltpu.SemaphoreType.DMA((NBUF,)),
        pltpu.SemaphoreType.DMA((NBUF,)),
    ],
    compiler_params=pltpu.CompilerParams(use_tc_tiling_on_sc=False),
)
def _gather_all(idxt_hbm, table_hbm, out_hbm, idx_v, rows_v, semg, sems):
    wid = lax.axis_index("s") * NC + lax.axis_index("c")
    base = wid * B_PER_W

    pltpu.sync_copy(idxt_hbm.at[:, pl.ds(base, B_PER_W)], idx_v)

    def group(g, carry):
        for b in range(NBUF):
            l = 2 * g + b // KPL
            k = b % KPL

            @pl.when(g > 0)
            def _wait_store():
                pltpu.make_async_copy(
                    rows_v.at[b], out_hbm.at[pl.ds(0, CHUNK), 0], sems.at[b]
                ).wait()

            pltpu.async_copy(
                table_hbm.at[idx_v.at[l, pl.ds(k * CHUNK, CHUNK)]],
                rows_v.at[b],
                semg.at[b],
            )
        for b in range(NBUF):
            l = 2 * g + b // KPL
            k = b % KPL
            pltpu.make_async_copy(
                table_hbm.at[idx_v.at[l, pl.ds(k * CHUNK, CHUNK)]],
                rows_v.at[b],
                semg.at[b],
            ).wait()
            pltpu.async_copy(
                rows_v.at[b],
                out_hbm.at[pl.ds(base + k * CHUNK, CHUNK), l],
                sems.at[b],
            )
        return carry

    lax.fori_loop(0, NGROUP, group, 0)

    for b in range(NBUF):
        pltpu.make_async_copy(
            rows_v.at[b], out_hbm.at[pl.ds(0, CHUNK), 0], sems.at[b]
        ).wait()


def kernel(input, table):
    return _gather_all(input.T, table)


# speedup vs baseline: 1.0527x; 1.0459x over previous
import functools

import jax
import jax.numpy as jnp
from jax import lax
from jax.experimental import pallas as pl
from jax.experimental.pallas import tpu as pltpu
from jax.experimental.pallas import tpu_sc as plsc

B = 16384
L = 50
EMBED = 64
NC, NS = 2, 16
NW = NC * NS
B_PER_W = B // NW
CHUNK = 128
KPL = B_PER_W // CHUNK
NBUF = 8
NGROUP = L * KPL // NBUF

_mesh = plsc.VectorSubcoreMesh(core_axis_name="c", subcore_axis_name="s")


@functools.partial(
    pl.kernel,
    mesh=_mesh,
    out_type=jax.ShapeDtypeStruct((L, B, EMBED), jnp.float32),
    scratch_types=[
        pltpu.VMEM((L, B_PER_W), jnp.int32),
        pltpu.VMEM((NBUF, CHUNK, EMBED), jnp.float32),
        pltpu.SemaphoreType.DMA((NBUF,)),
        pltpu.SemaphoreType.DMA((NBUF,)),
    ],
    compiler_params=pltpu.CompilerParams(use_tc_tiling_on_sc=False),
)
def _gather_all(idxt_hbm, table_hbm, out_hbm, idx_v, rows_v, semg, sems):
    wid = lax.axis_index("s") * NC + lax.axis_index("c")
    base = wid * B_PER_W

    pltpu.sync_copy(idxt_hbm.at[:, pl.ds(base, B_PER_W)], idx_v)

    def group(g, carry):
        for b in range(NBUF):
            l = 2 * g + b // KPL
            k = b % KPL

            @pl.when(g > 0)
            def _wait_store():
                pltpu.make_async_copy(
                    rows_v.at[b], out_hbm.at[0, pl.ds(0, CHUNK)], sems.at[b]
                ).wait()

            pltpu.async_copy(
                table_hbm.at[idx_v.at[l, pl.ds(k * CHUNK, CHUNK)]],
                rows_v.at[b],
                semg.at[b],
            )
        for b in range(NBUF):
            l = 2 * g + b // KPL
            k = b % KPL
            pltpu.make_async_copy(
                table_hbm.at[idx_v.at[l, pl.ds(k * CHUNK, CHUNK)]],
                rows_v.at[b],
                semg.at[b],
            ).wait()
            pltpu.async_copy(
                rows_v.at[b],
                out_hbm.at[l, pl.ds(base + k * CHUNK, CHUNK)],
                sems.at[b],
            )
        return carry

    lax.fori_loop(0, NGROUP, group, 0)

    for b in range(NBUF):
        pltpu.make_async_copy(
            rows_v.at[b], out_hbm.at[0, pl.ds(0, CHUNK)], sems.at[b]
        ).wait()


def kernel(input, table):
    return _gather_all(input.T, table).transpose(1, 0, 2)
